# baseline (speedup 1.0000x reference)
import functools

import jax
import jax.numpy as jnp
from jax import lax
from jax.experimental import pallas as pl
from jax.experimental.pallas import tpu as pltpu
from jax.experimental.pallas import tpu_sc as plsc

_D = 64
_SCALE = float(_D) ** 0.5
_NW = 32
_STREAM = 128
_K = 4
_CHUNK = _STREAM * _K


@functools.lru_cache(maxsize=None)
def _emb_kernel(n_total):
    per_w = n_total // _NW
    rows_per_w = per_w // _STREAM
    n_chunks = per_w // _CHUNK
    mesh = plsc.VectorSubcoreMesh(core_axis_name="c", subcore_axis_name="s")

    @functools.partial(
        pl.kernel,
        mesh=mesh,
        out_type=jax.ShapeDtypeStruct((n_total, _D), jnp.float32),
        scratch_types=[
            pltpu.VMEM((_K, _STREAM), jnp.int32),
            pltpu.VMEM((_CHUNK, _D), jnp.float32),
            pltpu.SemaphoreType.DMA,
        ],
        compiler_params=pltpu.CompilerParams(use_tc_tiling_on_sc=False),
    )
    def k(event_hbm, table_hbm, out_hbm, idx_v, rows_v, sem):
        wid = lax.axis_index("s") * 2 + lax.axis_index("c")
        row0 = wid * rows_per_w

        def chunk_body(g, carry):
            r = row0 + g * _K
            pltpu.sync_copy(event_hbm.at[pl.ds(r, _K)], idx_v)
            copies = [
                pltpu.async_copy(
                    table_hbm.at[idx_v.at[j]],
                    rows_v.at[pl.ds(j * _STREAM, _STREAM)],
                    sem,
                )
                for j in range(_K)
            ]
            for cp in copies:
                cp.wait()

            def scale_row(rr, c2):
                for c in range(_D // 16):
                    sl = (rr, pl.ds(c * 16, 16))
                    rows_v[sl] = rows_v[sl] * _SCALE
                return c2

            lax.fori_loop(0, _CHUNK, scale_row, 0)
            pltpu.sync_copy(rows_v, out_hbm.at[pl.ds(r * _STREAM, _CHUNK)])
            return carry

        lax.fori_loop(0, n_chunks, chunk_body, 0)

    return k


def kernel(event, table):
    n = event.size
    ev = event.reshape(n // _STREAM, _STREAM).astype(jnp.int32)
    out = _emb_kernel(n)(ev, table)
    return out.reshape(*event.shape, _D)


# speedup vs baseline: 1.2151x; 1.2151x over previous
import functools

import jax
import jax.numpy as jnp
from jax import lax
from jax.experimental import pallas as pl
from jax.experimental.pallas import tpu as pltpu
from jax.experimental.pallas import tpu_sc as plsc

_D = 64
_SCALE = float(_D) ** 0.5
_NW = 32
_STREAM = 128
_K = 2
_CHUNK = _STREAM * _K
_NBUF = 4


@functools.lru_cache(maxsize=None)
def _emb_kernel(n_total):
    per_w = n_total // _NW
    rows_per_w = per_w // _STREAM
    n_chunks = per_w // _CHUNK
    n_groups = n_chunks // _NBUF
    assert n_chunks % _NBUF == 0 and n_groups >= 3
    mesh = plsc.VectorSubcoreMesh(core_axis_name="c", subcore_axis_name="s")

    @functools.partial(
        pl.kernel,
        mesh=mesh,
        out_type=jax.ShapeDtypeStruct((n_total, _D), jnp.float32),
        scratch_types=[
            pltpu.VMEM((_NBUF, _K, _STREAM), jnp.int32),
            pltpu.VMEM((_NBUF, _CHUNK, _D), jnp.float32),
            pltpu.SemaphoreType.DMA((_NBUF,)),
            pltpu.SemaphoreType.DMA((_NBUF,)),
            pltpu.SemaphoreType.DMA((_NBUF,)),
        ],
        compiler_params=pltpu.CompilerParams(use_tc_tiling_on_sc=False),
    )
    def k(event_hbm, table_hbm, out_hbm, idx_v, rows_v, sem_i, sem_g, sem_o):
        wid = lax.axis_index("s") * 2 + lax.axis_index("c")
        row0 = wid * rows_per_w

        def launch_gather(b):
            for j in range(_K):
                pltpu.async_copy(
                    table_hbm.at[idx_v.at[b].at[j]],
                    rows_v.at[b].at[pl.ds(j * _STREAM, _STREAM)],
                    sem_g.at[b],
                )

        def launch_idx(g, b):
            pltpu.async_copy(
                event_hbm.at[pl.ds(row0 + g * _K, _K)], idx_v.at[b], sem_i.at[b]
            )

        def wait_idx(b):
            pltpu.make_async_copy(
                event_hbm.at[pl.ds(0, _K)], idx_v.at[b], sem_i.at[b]
            ).wait()

        def wait_gather(b):
            pltpu.make_async_copy(
                table_hbm.at[pl.ds(0, _CHUNK)], rows_v.at[b], sem_g.at[b]
            ).wait()

        def wait_out(b):
            pltpu.make_async_copy(
                rows_v.at[b], out_hbm.at[pl.ds(0, _CHUNK)], sem_o.at[b]
            ).wait()

        def half(g, b, first_group=False, last_group=False):
            bn = (b + 1) % _NBUF
            bi = (b + 2) % _NBUF
            if not (first_group and b < _NBUF - 1):
                wait_out(bn)
            if not (last_group and b == _NBUF - 1):
                wait_idx(bn)
                launch_gather(bn)
            wait_gather(b)
            if not (last_group and b >= _NBUF - 2):
                launch_idx(g + 2, bi)
            rows = rows_v.at[b]

            def scale_row(rr, c2):
                for c in range(_D // 16):
                    sl = (rr, pl.ds(c * 16, 16))
                    rows[sl] = rows[sl] * _SCALE
                return c2

            lax.fori_loop(0, _CHUNK, scale_row, 0)
            pltpu.async_copy(
                rows_v.at[b],
                out_hbm.at[pl.ds((row0 + g * _K) * _STREAM, _CHUNK)],
                sem_o.at[b],
            )

        cp0 = pltpu.async_copy(
            event_hbm.at[pl.ds(row0, _K)], idx_v.at[0], sem_i.at[0]
        )
        launch_idx(1, 1)
        cp0.wait()
        launch_gather(0)

        for b in range(_NBUF):
            half(b, b, first_group=True)

        def group(gi, carry):
            for b in range(_NBUF):
                half(gi * _NBUF + b, b)
            return carry

        lax.fori_loop(1, n_groups - 1, group, 0)

        for b in range(_NBUF):
            half((n_groups - 1) * _NBUF + b, b, last_group=True)

        for b in range(1, _NBUF):
            wait_out(b)

    return k


def kernel(event, table):
    n = event.size
    ev = event.reshape(n // _STREAM, _STREAM).astype(jnp.int32)
    out = _emb_kernel(n)(ev, table)
    return out.reshape(*event.shape, _D)
